# baseline (speedup 1.0000x reference)
import dataclasses

import jax
import jax.numpy as jnp
from jax import lax
from jax.experimental import pallas as pl
from jax.experimental.pallas import tpu as pltpu
from jax.experimental.pallas import tpu_sc as plsc

_N = 10000
_NP = 10240
_E = 320000
_G = 64
_F = 128
_NW = 32
_R = _NP // _NW
_EW = _E // _NW
_CH = 128
_NCH = 88
_S = _CH * _NCH
_CP = 80
_HI = jax.lax.Precision.HIGHEST

_SC_PARAMS = pltpu.CompilerParams()
if "needs_layout_passes" in pltpu.CompilerParams.__dataclass_fields__:
    _SC_PARAMS = dataclasses.replace(_SC_PARAMS, needs_layout_passes=False)

def _mesh_kw():
    return dict(mesh=plsc.VectorSubcoreMesh(core_axis_name="c",
                                            subcore_axis_name="s"),
                compiler_params=_SC_PARAMS)


def _lrelu(x):
    return jnp.where(x >= 0, x, 0.2 * x)


def _wid():
    return lax.axis_index("c") * 16 + lax.axis_index("s")


def _count_body(pk_hbm, cnt_out, pk_v, cm_v, cnt_v):
    w = _wid()
    pltpu.sync_copy(pk_hbm.at[w], pk_v)

    z16 = jnp.zeros((16,), jnp.int32)
    for r in range(16):
        cm_v[r, pl.ds(0, 16)] = z16
        cm_v[r, pl.ds(16, 16)] = z16

    i16 = lax.iota(jnp.int32, 16)
    one16 = jnp.ones((16,), jnp.int32)

    @pl.loop(0, _EW // 16)
    def _(g):
        pk16 = pk_v[pl.ds(g * 16, 16)]
        bv = jnp.right_shift(pk16, 16) // _R
        plsc.addupdate_scatter(cm_v, [i16, bv], one16)

    for c in range(2):
        sl = pl.ds(c * 16, 16)
        acc = jnp.zeros((16,), jnp.int32)
        for r in range(16):
            acc = acc + cm_v[r, sl]
        cnt_v[sl] = acc
    pltpu.sync_copy(cnt_v, cnt_out.at[pl.ds(w * 32, 32)])


def _prep_count(pk2):
    kern = pl.kernel(
        _count_body,
        out_type=jax.ShapeDtypeStruct((_NW * _NW,), jnp.int32),
        scratch_types=[
            pltpu.VMEM((_EW,), jnp.int32),
            pltpu.VMEM((16, 32), jnp.int32),
            pltpu.VMEM((32,), jnp.int32),
        ],
        **_mesh_kw(),
    )
    return kern(pk2)


def _bucket_body(pk_hbm, arr_hbm, pks_out, pk_v, loc_v, vb_v, pb_v):
    w = _wid()
    pltpu.sync_copy(pk_hbm.at[w], pk_v)
    pltpu.sync_copy(arr_hbm.at[pl.ds(w * 32, 32)], loc_v)

    @pl.loop(0, _EW // _CP)
    def _(ci):
        for g in range(_CP // 16):
            sl = pl.ds(g * 16, 16)
            pk16 = pk_v[pl.ds(ci * _CP + g * 16, 16)]
            bv = jnp.right_shift(pk16, 16) // _R
            rank, islast = plsc.scan_count(bv)
            rank = rank - jnp.full((16,), rank[0], jnp.int32)
            cur = plsc.load_gather(loc_v, [bv])
            vb_v[0, sl] = pk16
            pb_v[0, sl] = cur + rank
            plsc.addupdate_scatter(loc_v, [bv], rank + 1, mask=islast)
        pltpu.sync_copy(vb_v.at[0], pks_out.at[pb_v.at[0]])


def _prep_scatter(pk2, arr):
    kern = pl.kernel(
        _bucket_body,
        out_type=jax.ShapeDtypeStruct((_NW * _S,), jnp.int32),
        scratch_types=[
            pltpu.VMEM((_EW,), jnp.int32),
            pltpu.VMEM((32,), jnp.int32),
            pltpu.VMEM((1, _CP), jnp.int32),
            pltpu.VMEM((1, _CP), jnp.int32),
        ],
        **_mesh_kw(),
    )
    return kern(pk2, arr)


def _edge_body(h_hbm, ev_hbm, m_hbm, pks_hbm, arr_hbm, p_out, den_out,
               ev_v, m_v, cnt_v, pk_c, si_c, di_c, pv_c, rows_v,
               acc_l, den_l, si0, si1, sg0, sg1):
    w = _wid()
    sem_i = (si0, si1)
    sem_g = (sg0, sg1)
    base_slot = w * _S
    z16 = jnp.zeros((16,), jnp.float32)

    @pl.loop(0, _R + 8)
    def _(r):
        for c in range(8):
            acc_l[r, pl.ds(c * 16, 16)] = z16

    @pl.loop(0, (_R + 32) // 16)
    def _(j):
        den_l[pl.ds(j * 16, 16)] = z16

    pltpu.sync_copy(ev_hbm, ev_v)
    pltpu.sync_copy(m_hbm.at[0, pl.ds(0, 16)], m_v)
    pltpu.sync_copy(arr_hbm.at[pl.ds(_NW * 32, 32)], cnt_v)

    m16 = m_v[...]
    i16 = lax.iota(jnp.int32, 16)
    oh0 = (i16 == 0).astype(jnp.float32)
    msk = jnp.full((16,), 0xFFFF, jnp.int32)
    cntv = plsc.load_gather(cnt_v, [jnp.full((16,), w, jnp.int32)])
    cnt_t = cntv[0]
    dump = jnp.full((16,), _R, jnp.int32) + (i16 & 7)

    for b in range(2):
        pltpu.async_copy(pks_hbm.at[pl.ds(base_slot + b * _CH, _CH)],
                         pk_c.at[b], sem_i[b])

    @pl.loop(0, _NCH // 2)
    def _(it):
        for b in range(2):
            ci = 2 * it + b
            start = ci * _CH
            pltpu.make_async_copy(
                pks_hbm.at[pl.ds(base_slot + start, _CH)], pk_c.at[b],
                sem_i[b]).wait()

            @pl.when(it < _NCH // 2 - 1)
            def _():
                pltpu.async_copy(
                    pks_hbm.at[pl.ds(base_slot + start + 2 * _CH, _CH)],
                    pk_c.at[b], sem_i[b])

            @pl.when(start < cnt_t)
            def _():
                @pl.loop(0, _CH // 16)
                def _(g):
                    sl = pl.ds(g * 16, 16)
                    pk16 = pk_c[b, sl]
                    sv = jnp.minimum(pk16 & msk, _NP - 1)
                    dvg = jnp.right_shift(pk16, 16)
                    dvc = jnp.clip(dvg, 0, _NP - 1)
                    valid = (start + g * 16 + i16) < cntv
                    dl = jnp.where(valid, dvg - w * _R, dump)
                    si_c[b, sl] = sv
                    di_c[b, sl] = dl
                    e = (plsc.load_gather(ev_v, [sv + sv])
                         + plsc.load_gather(ev_v, [dvc + dvc + 1]))
                    e = jnp.where(e >= 0.0, e, 0.2 * e)
                    p = jnp.exp(e - m16)
                    pv_c[b, sl] = jnp.where(valid, p, 0.0)

                pltpu.async_copy(h_hbm.at[si_c.at[b]], rows_v.at[b],
                                 sem_g[b])
                pltpu.make_async_copy(h_hbm.at[si_c.at[b]], rows_v.at[b],
                                      sem_g[b]).wait()

                b16 = jnp.full((16,), b, jnp.int32)

                @pl.loop(0, _CH)
                def _(r):
                    r16 = jnp.full((16,), r, jnp.int32)
                    dl = plsc.load_gather(di_c, [b16, r16])[0]
                    pb = plsc.load_gather(pv_c, [b16, r16])
                    for c in range(8):
                        sl = pl.ds(c * 16, 16)
                        acc_l[dl, sl] = (acc_l[dl, sl]
                                         + pb * rows_v[b, r, sl])
                    dsl = pl.ds(dl, 16)
                    den_l[dsl] = den_l[dsl] + pb * oh0

    out_rows = pl.ds(w * _R, _R)
    pltpu.sync_copy(acc_l.at[pl.ds(0, _R)], p_out.at[out_rows])
    pltpu.sync_copy(den_l.at[pl.ds(0, _R)], den_out.at[out_rows])


def _sc_edge(h, ev, m, pks, arr):
    kern = pl.kernel(
        _edge_body,
        out_type=[jax.ShapeDtypeStruct((_NP, _F), jnp.float32),
                  jax.ShapeDtypeStruct((_NP,), jnp.float32)],
        scratch_types=[
            pltpu.VMEM((2 * _NP,), jnp.float32),
            pltpu.VMEM((16,), jnp.float32),
            pltpu.VMEM((32,), jnp.int32),
            pltpu.VMEM((2, _CH), jnp.int32),
            pltpu.VMEM((2, _CH), jnp.int32),
            pltpu.VMEM((2, _CH), jnp.int32),
            pltpu.VMEM((2, _CH), jnp.float32),
            pltpu.VMEM((2, _CH, _F), jnp.float32),
            pltpu.VMEM((_R + 8, _F), jnp.float32),
            pltpu.VMEM((_R + 32,), jnp.float32),
        ] + [pltpu.SemaphoreType.DMA] * 4,
        **_mesh_kw(),
    )
    return kern(h, ev, m, pks, arr)


def _offsets(cnt_ref):
    cf = cnt_ref[...].astype(jnp.float32)
    ii = lax.broadcasted_iota(jnp.int32, (_NW, _NW), 0)
    jj = lax.broadcasted_iota(jnp.int32, (_NW, _NW), 1)
    lower = (jj < ii).astype(jnp.float32)
    pre = jnp.dot(lower, cf, precision=_HI)
    off = pre + (jj * _S).astype(jnp.float32)
    tot = jnp.sum(cf, axis=0, keepdims=True)
    return jnp.concatenate([off, tot], axis=0).astype(jnp.int32)


def _pre_body(x_ref, w_ref, av_ref, cnt_ref, h_ref, ev_ref, m_ref, arr_ref):
    h = jnp.dot(x_ref[...], w_ref[...], precision=_HI)
    h_ref[...] = h
    ev = jnp.dot(h, av_ref[...], precision=_HI)
    ev_ref[...] = ev
    mx = jnp.max(ev, axis=0, keepdims=True)
    m = _lrelu(mx[0:1, 0:1] + mx[0:1, 1:2])
    m_ref[...] = jnp.broadcast_to(m, (8, 128))
    arr_ref[...] = _offsets(cnt_ref)


def _tc_pre(x, w, av, cnt):
    return pl.pallas_call(
        _pre_body,
        out_shape=[jax.ShapeDtypeStruct((_NP, _F), jnp.float32),
                   jax.ShapeDtypeStruct((_NP, 2), jnp.float32),
                   jax.ShapeDtypeStruct((8, 128), jnp.float32),
                   jax.ShapeDtypeStruct((_NW + 1, _NW), jnp.int32)],
    )(x, w, av, cnt)


def _combine(p_ref, d_ref, ev_ref, m_ref, h_ref, b_ref):
    es = ev_ref[:, 0:1]
    ed = ev_ref[:, 1:2]
    m = m_ref[0:1, 0:1]
    ps = jnp.exp(_lrelu(es + ed) - m)
    h = h_ref[...]
    num = p_ref[...] + ps * h
    den = d_ref[...] + ps + 1e-16
    return num / den + b_ref[...]


def _mid_body(p_ref, d_ref, ev_ref, m_ref, h_ref, b_ref, w_ref, av_ref,
              h2_ref, ev2_ref, m2_ref):
    a = jnp.maximum(_combine(p_ref, d_ref, ev_ref, m_ref, h_ref, b_ref), 0.0)
    h2 = jnp.dot(a, w_ref[...], precision=_HI)
    h2_ref[...] = h2
    ev2 = jnp.dot(h2, av_ref[...], precision=_HI)
    ev2_ref[...] = ev2
    mx = jnp.max(ev2, axis=0, keepdims=True)
    m2 = _lrelu(mx[0:1, 0:1] + mx[0:1, 1:2])
    m2_ref[...] = jnp.broadcast_to(m2, (8, 128))


def _tc_mid(p, d, ev, m, h, b, w, av):
    return pl.pallas_call(
        _mid_body,
        out_shape=[jax.ShapeDtypeStruct((_NP, _F), jnp.float32),
                   jax.ShapeDtypeStruct((_NP, 2), jnp.float32),
                   jax.ShapeDtypeStruct((8, 128), jnp.float32)],
    )(p, d, ev, m, h, b, w, av)


def _fin_body(p_ref, d_ref, ev_ref, m_ref, h_ref, b_ref, batch_ref,
              wl_ref, bl_ref, out_ref):
    o = _combine(p_ref, d_ref, ev_ref, m_ref, h_ref, b_ref)
    bb = jnp.broadcast_to(batch_ref[...], (_G, _NP))
    gid = lax.broadcasted_iota(jnp.int32, (_G, _NP), 0)
    oh = (bb == gid).astype(jnp.float32)
    sums = jnp.dot(oh, o, precision=_HI)
    cnt = jnp.sum(oh, axis=1, keepdims=True)
    pooled = sums / jnp.maximum(cnt, 1.0)
    out_ref[...] = jnp.dot(pooled, wl_ref[...], precision=_HI) + bl_ref[...]


def _tc_fin(p, d, ev, m, h, b, batch2, wl, bl):
    return pl.pallas_call(
        _fin_body,
        out_shape=jax.ShapeDtypeStruct((_G, _F), jnp.float32),
    )(p, d, ev, m, h, b, batch2, wl, bl)


def kernel(x, edge_index, batch, W1, a_src1, a_dst1, b1, W2, a_src2, a_dst2,
           b2, W3, a_src3, a_dst3, b3, Wlin, blin):
    x_p = jnp.pad(x, ((0, _NP - _N), (0, 0)))
    pk2 = (edge_index[0] | (edge_index[1] << 16)).reshape(_NW, _EW)
    batch2 = jnp.concatenate(
        [batch, jnp.full((_NP - _N,), _G, jnp.int32)]).reshape(1, _NP)

    av1 = jnp.stack([a_src1, a_dst1], axis=1)
    av2 = jnp.stack([a_src2, a_dst2], axis=1)
    av3 = jnp.stack([a_src3, a_dst3], axis=1)

    counts = _prep_count(pk2)
    h1, ev1, m1, arr = _tc_pre(x_p, W1, av1, counts.reshape(_NW, _NW))
    arr = arr.reshape((_NW + 1) * _NW)
    pks = _prep_scatter(pk2, arr)

    P1, D1 = _sc_edge(h1, ev1.reshape(2 * _NP), m1, pks, arr)
    h2, ev2, m2 = _tc_mid(P1, D1.reshape(_NP, 1), ev1, m1, h1,
                          b1.reshape(1, _F), W2, av2)
    P2, D2 = _sc_edge(h2, ev2.reshape(2 * _NP), m2, pks, arr)
    h3, ev3, m3 = _tc_mid(P2, D2.reshape(_NP, 1), ev2, m2, h2,
                          b2.reshape(1, _F), W3, av3)
    P3, D3 = _sc_edge(h3, ev3.reshape(2 * _NP), m3, pks, arr)
    return _tc_fin(P3, D3.reshape(_NP, 1), ev3, m3, h3,
                   b3.reshape(1, _F), batch2, Wlin, blin.reshape(1, _F))


# speedup vs baseline: 1.8201x; 1.8201x over previous
import dataclasses

import jax
import jax.numpy as jnp
from jax import lax
from jax.experimental import pallas as pl
from jax.experimental.pallas import tpu as pltpu
from jax.experimental.pallas import tpu_sc as plsc

_N = 10000
_NP = 10240
_E = 320000
_G = 64
_F = 128
_NW = 32
_CH = 128
_NCH = 80
_EP = _NW * _NCH * _CH
_DUMP = 16
_AR = _NP + _DUMP
_HI = jax.lax.Precision.HIGHEST


def _lrelu(x):
    return jnp.where(x >= 0, x, 0.2 * x)


def _prep_p_body(ev_hbm, m_hbm, pk_hbm, p_out, ev_v, m_v, pk_v, pb_v):
    cid = lax.axis_index("c")
    sid = lax.axis_index("s")
    wid = cid * 16 + sid
    ew = _NCH * _CH
    base = wid * ew
    pltpu.sync_copy(ev_hbm, ev_v)
    pltpu.sync_copy(m_hbm.at[0, pl.ds(0, 16)], m_v)
    pltpu.sync_copy(pk_hbm.at[pl.ds(base, ew)], pk_v)

    m16 = m_v[...]
    one16 = jnp.ones((16,), jnp.int32)
    msk = jnp.full((16,), 0xFFFF, jnp.int32)

    @pl.loop(0, ew // 16)
    def _(g):
        sl = pl.ds(g * 16, 16)
        pk16 = pk_v[sl]
        iv = pk16 & msk
        dv = jnp.minimum(jnp.right_shift(pk16, 16), _NP - 1)
        e = (plsc.load_gather(ev_v, [iv + iv])
             + plsc.load_gather(ev_v, [dv + dv + one16]))
        e = jnp.where(e >= 0.0, e, 0.2 * e)
        pb_v[sl] = jnp.exp(e - m16)

    pltpu.sync_copy(pb_v, p_out.at[pl.ds(base, ew)])


def _prep_p(ev, m, pk):
    kern = pl.kernel(
        _prep_p_body,
        compiler_params=_SC_PARAMS,
        out_type=jax.ShapeDtypeStruct((_EP,), jnp.float32),
        mesh=plsc.VectorSubcoreMesh(core_axis_name="c", subcore_axis_name="s"),
        scratch_types=[
            pltpu.VMEM((2 * _NP,), jnp.float32),
            pltpu.VMEM((16,), jnp.float32),
            pltpu.VMEM((_NCH * _CH,), jnp.int32),
            pltpu.VMEM((_NCH * _CH,), jnp.float32),
        ],
    )
    return kern(ev, m, pk)


def _edge_body(h_hbm, pk_hbm, pv_hbm, p_out, den_out,
               pk_c, pv_c, si_c, di_c, rows_v, acc_sh, den_sh,
               sk0, sk1, sp0, sp1, sg0, sg1, ss0, ss1, sd0, sd1):
    cid = lax.axis_index("c")
    sid = lax.axis_index("s")
    wid = cid * 16 + sid
    base = sid * (_NP // 16)
    ebase = wid * _NCH * _CH
    sem_k = (sk0, sk1)
    sem_p = (sp0, sp1)
    sem_g = (sg0, sg1)
    sem_s = (ss0, ss1)
    sem_d = (sd0, sd1)

    z16 = jnp.zeros((16,), jnp.float32)

    @pl.loop(0, _CH)
    def _(r):
        for b in range(2):
            for c in range(8):
                rows_v[b, r, pl.ds(c * 16, 16)] = z16

    for b in range(2):
        @pl.loop(0, _CH // 16)
        def _(j):
            pv_c[b, pl.ds(j * 16, 16)] = z16

    for k in range(_NP // 16 // _CH):
        pltpu.sync_copy(rows_v.at[0], acc_sh.at[pl.ds(base + k * _CH, _CH)])
        pltpu.sync_copy(pv_c.at[0], den_sh.at[pl.ds(base + k * _CH, _CH)])

    @pl.when(sid == 0)
    def _():
        pltpu.sync_copy(rows_v.at[0, pl.ds(0, _DUMP)],
                        acc_sh.at[pl.ds(_NP, _DUMP)])
        pltpu.sync_copy(pv_c.at[0, pl.ds(0, _DUMP)],
                        den_sh.at[pl.ds(_NP, _DUMP)])

    plsc.subcore_barrier()

    msk = jnp.full((16,), 0xFFFF, jnp.int32)

    for b in range(2):
        pltpu.async_copy(pk_hbm.at[pl.ds(ebase + b * _CH, _CH)],
                         pk_c.at[b], sem_k[b])
    pltpu.async_copy(pv_hbm.at[pl.ds(ebase, _CH)], pv_c.at[0], sem_p[0])

    def scale_and_scatter(q, ci_q):
        pltpu.make_async_copy(h_hbm.at[si_c.at[q]], rows_v.at[q],
                              sem_g[q]).wait()
        pltpu.make_async_copy(pv_hbm.at[pl.ds(ebase + ci_q * _CH, _CH)],
                              pv_c.at[q], sem_p[q]).wait()
        q16 = jnp.full((16,), q, jnp.int32)

        @pl.loop(0, _CH // 4)
        def _(i):
            for u in range(4):
                r = i * 4 + u
                pb = plsc.load_gather(pv_c, [q16, jnp.full((16,), r,
                                                           jnp.int32)])
                for c in range(8):
                    sl = pl.ds(c * 16, 16)
                    rows_v[q, r, sl] = rows_v[q, r, sl] * pb

        pltpu.async_copy(rows_v.at[q], acc_sh.at[di_c.at[q]],
                         sem_s[q], add=True)
        pltpu.async_copy(pv_c.at[q], den_sh.at[di_c.at[q]],
                         sem_d[q], add=True)

    @pl.loop(0, _NCH // 2)
    def _(it):
        for b in range(2):
            ci = 2 * it + b
            sl_c = pl.ds(ebase + ci * _CH, _CH)
            pltpu.make_async_copy(pk_hbm.at[sl_c], pk_c.at[b],
                                  sem_k[b]).wait()

            @pl.when(it > 0)
            def _():
                pltpu.make_async_copy(rows_v.at[b], acc_sh.at[di_c.at[b]],
                                      sem_s[b]).wait()
                pltpu.make_async_copy(pv_c.at[b], den_sh.at[di_c.at[b]],
                                      sem_d[b]).wait()

            @pl.loop(0, _CH // 16)
            def _(j):
                sl = pl.ds(j * 16, 16)
                pk16 = pk_c[b, sl]
                si_c[b, sl] = pk16 & msk
                di_c[b, sl] = jnp.right_shift(pk16, 16)

            pltpu.async_copy(h_hbm.at[si_c.at[b]], rows_v.at[b], sem_g[b])

            @pl.when(ci < _NCH - 2)
            def _():
                sl_n = pl.ds(ebase + (ci + 2) * _CH, _CH)
                pltpu.async_copy(pk_hbm.at[sl_n], pk_c.at[b], sem_k[b])

            @pl.when(ci > 0)
            def _():
                scale_and_scatter(1 - b, ci - 1)

            @pl.when(ci < _NCH - 1)
            def _():
                pltpu.async_copy(
                    pv_hbm.at[pl.ds(ebase + (ci + 1) * _CH, _CH)],
                    pv_c.at[1 - b], sem_p[1 - b])

    scale_and_scatter(1, _NCH - 1)

    for b in range(2):
        pltpu.make_async_copy(rows_v.at[b], acc_sh.at[di_c.at[b]],
                              sem_s[b]).wait()
        pltpu.make_async_copy(pv_c.at[b], den_sh.at[di_c.at[b]],
                              sem_d[b]).wait()

    plsc.subcore_barrier()
    out_sl = pl.ds(base, _NP // 16)
    pltpu.sync_copy(acc_sh.at[out_sl], p_out.at[cid, out_sl])
    pltpu.sync_copy(den_sh.at[out_sl], den_out.at[cid, out_sl])


_SC_PARAMS = pltpu.CompilerParams()
if "needs_layout_passes" in pltpu.CompilerParams.__dataclass_fields__:
    _SC_PARAMS = dataclasses.replace(_SC_PARAMS, needs_layout_passes=False)


def _sc_edge(h, pk, pv):
    kern = pl.kernel(
        _edge_body,
        compiler_params=_SC_PARAMS,
        out_type=[jax.ShapeDtypeStruct((2, _NP, _F), jnp.float32),
                  jax.ShapeDtypeStruct((2, _NP), jnp.float32)],
        mesh=plsc.VectorSubcoreMesh(core_axis_name="c", subcore_axis_name="s"),
        scratch_types=[
            pltpu.VMEM((2, _CH), jnp.int32),
            pltpu.VMEM((2, _CH), jnp.float32),
            pltpu.VMEM((2, _CH), jnp.int32),
            pltpu.VMEM((2, _CH), jnp.int32),
            pltpu.VMEM((2, _CH, _F), jnp.float32),
            pltpu.VMEM_SHARED((_AR, _F), jnp.float32),
            pltpu.VMEM_SHARED((_AR,), jnp.float32),
        ] + [pltpu.SemaphoreType.DMA] * 10,
    )
    return kern(h, pk, pv)


def _pre_body(x_ref, w_ref, av_ref, h_ref, ev_ref, m_ref):
    h = jnp.dot(x_ref[...], w_ref[...], precision=_HI)
    h_ref[...] = h
    ev = jnp.dot(h, av_ref[...], precision=_HI)
    ev_ref[...] = ev
    mx = jnp.max(ev, axis=0, keepdims=True)
    m = _lrelu(mx[0:1, 0:1] + mx[0:1, 1:2])
    m_ref[...] = jnp.broadcast_to(m, (8, 128))


def _tc_pre(x, w, av):
    return pl.pallas_call(
        _pre_body,
        out_shape=[jax.ShapeDtypeStruct((_NP, _F), jnp.float32),
                   jax.ShapeDtypeStruct((_NP, 2), jnp.float32),
                   jax.ShapeDtypeStruct((8, 128), jnp.float32)],
    )(x, w, av)


def _combine(p_ref, d_ref, ev_ref, m_ref, h_ref, b_ref):
    es = ev_ref[:, 0:1]
    ed = ev_ref[:, 1:2]
    m = m_ref[0:1, 0:1]
    ps = jnp.exp(_lrelu(es + ed) - m)
    h = h_ref[...]
    num = p_ref[0] + p_ref[1] + ps * h
    den = d_ref[0] + d_ref[1] + ps + 1e-16
    return num / den + b_ref[...]


def _mid_body(p_ref, d_ref, ev_ref, m_ref, h_ref, b_ref, w_ref, av_ref,
              h2_ref, ev2_ref, m2_ref):
    a = jnp.maximum(_combine(p_ref, d_ref, ev_ref, m_ref, h_ref, b_ref), 0.0)
    h2 = jnp.dot(a, w_ref[...], precision=_HI)
    h2_ref[...] = h2
    ev2 = jnp.dot(h2, av_ref[...], precision=_HI)
    ev2_ref[...] = ev2
    mx = jnp.max(ev2, axis=0, keepdims=True)
    m2 = _lrelu(mx[0:1, 0:1] + mx[0:1, 1:2])
    m2_ref[...] = jnp.broadcast_to(m2, (8, 128))


def _tc_mid(p, d, ev, m, h, b, w, av):
    return pl.pallas_call(
        _mid_body,
        out_shape=[jax.ShapeDtypeStruct((_NP, _F), jnp.float32),
                   jax.ShapeDtypeStruct((_NP, 2), jnp.float32),
                   jax.ShapeDtypeStruct((8, 128), jnp.float32)],
    )(p, d, ev, m, h, b, w, av)


def _fin_body(p_ref, d_ref, ev_ref, m_ref, h_ref, b_ref, batch_ref,
              wl_ref, bl_ref, out_ref):
    o = _combine(p_ref, d_ref, ev_ref, m_ref, h_ref, b_ref)
    bb = jnp.broadcast_to(batch_ref[...], (_G, _NP))
    gid = lax.broadcasted_iota(jnp.int32, (_G, _NP), 0)
    oh = (bb == gid).astype(jnp.float32)
    sums = jnp.dot(oh, o, precision=_HI)
    cnt = jnp.sum(oh, axis=1, keepdims=True)
    pooled = sums / jnp.maximum(cnt, 1.0)
    out_ref[...] = jnp.dot(pooled, wl_ref[...], precision=_HI) + bl_ref[...]


def _tc_fin(p, d, ev, m, h, b, batch2, wl, bl):
    return pl.pallas_call(
        _fin_body,
        out_shape=jax.ShapeDtypeStruct((_G, _F), jnp.float32),
    )(p, d, ev, m, h, b, batch2, wl, bl)


def kernel(x, edge_index, batch, W1, a_src1, a_dst1, b1, W2, a_src2, a_dst2,
           b2, W3, a_src3, a_dst3, b3, Wlin, blin):
    x_p = jnp.pad(x, ((0, _NP - _N), (0, 0)))
    src = edge_index[0]
    dst = edge_index[1]
    npad = _EP - _E
    src_p = jnp.concatenate([src, jnp.zeros((npad,), jnp.int32)])
    dst_p = jnp.concatenate(
        [dst, _NP + (jnp.arange(npad, dtype=jnp.int32) % _DUMP)])
    pk1 = src_p | (dst_p << 16)
    batch2 = jnp.concatenate(
        [batch, jnp.full((_NP - _N,), _G, jnp.int32)]).reshape(1, _NP)

    av1 = jnp.stack([a_src1, a_dst1], axis=1)
    av2 = jnp.stack([a_src2, a_dst2], axis=1)
    av3 = jnp.stack([a_src3, a_dst3], axis=1)

    h1, ev1, m1 = _tc_pre(x_p, W1, av1)
    pv1 = _prep_p(ev1.reshape(2 * _NP), m1, pk1)
    P1, D1 = _sc_edge(h1, pk1, pv1)
    h2, ev2, m2 = _tc_mid(P1, D1.reshape(2, _NP, 1), ev1, m1, h1,
                          b1.reshape(1, _F), W2, av2)
    pv2 = _prep_p(ev2.reshape(2 * _NP), m2, pk1)
    P2, D2 = _sc_edge(h2, pk1, pv2)
    h3, ev3, m3 = _tc_mid(P2, D2.reshape(2, _NP, 1), ev2, m2, h2,
                          b2.reshape(1, _F), W3, av3)
    pv3 = _prep_p(ev3.reshape(2 * _NP), m3, pk1)
    P3, D3 = _sc_edge(h3, pk1, pv3)
    return _tc_fin(P3, D3.reshape(2, _NP, 1), ev3, m3, h3,
                   b3.reshape(1, _F), batch2, Wlin, blin.reshape(1, _F))


# speedup vs baseline: 1.9989x; 1.0983x over previous
import dataclasses

import jax
import jax.numpy as jnp
from jax import lax
from jax.experimental import pallas as pl
from jax.experimental.pallas import tpu as pltpu
from jax.experimental.pallas import tpu_sc as plsc

_N = 10000
_NP = 10240
_E = 320000
_G = 64
_F = 128
_NW = 32
_CH = 64
_NCH = 158
_EP = _NW * _NCH * _CH
_DUMP = 16
_AR = _NP + _DUMP
_HI = jax.lax.Precision.HIGHEST


def _lrelu(x):
    return jnp.where(x >= 0, x, 0.2 * x)


def _edge_body(h_hbm, ev_hbm, m_hbm, pk_hbm, p_out, den_out,
               ev_v, m_v, pk_c, si_c, di_c, rows_v, pv_v, acc_sh, den_sh,
               si0, si1, sg0, sg1, ss0, ss1, sd0, sd1):
    cid = lax.axis_index("c")
    sid = lax.axis_index("s")
    wid = cid * 16 + sid
    base = sid * (_NP // 16)
    sem_i = (si0, si1)
    sem_g = (sg0, sg1)
    sem_s = (ss0, ss1)
    sem_d = (sd0, sd1)

    z16 = jnp.zeros((16,), jnp.float32)

    @pl.loop(0, _CH)
    def _(r):
        for b in range(2):
            for c in range(8):
                rows_v[b, r, pl.ds(c * 16, 16)] = z16

    for b in range(2):
        @pl.loop(0, _CH // 16)
        def _(j):
            pv_v[b, pl.ds(j * 16, 16)] = z16

    for k in range(_NP // 16 // _CH):
        bb = k % 2
        pltpu.sync_copy(rows_v.at[bb], acc_sh.at[pl.ds(base + k * _CH, _CH)])
        pltpu.sync_copy(pv_v.at[bb], den_sh.at[pl.ds(base + k * _CH, _CH)])

    @pl.when(sid == 0)
    def _():
        pltpu.sync_copy(rows_v.at[0, pl.ds(0, _DUMP)],
                        acc_sh.at[pl.ds(_NP, _DUMP)])
        pltpu.sync_copy(pv_v.at[0, pl.ds(0, _DUMP)],
                        den_sh.at[pl.ds(_NP, _DUMP)])

    pltpu.sync_copy(ev_hbm, ev_v)
    pltpu.sync_copy(m_hbm.at[0, pl.ds(0, 16)], m_v)
    plsc.subcore_barrier()

    m16 = m_v[...]
    one16 = jnp.ones((16,), jnp.int32)
    msk = jnp.full((16,), 0xFFFF, jnp.int32)

    for b in range(2):
        pltpu.async_copy(pk_hbm.at[wid, b], pk_c.at[b], sem_i[b])

    @pl.loop(0, _NCH // 2)
    def _(it):
        for b in range(2):
            ci = 2 * it + b
            pltpu.make_async_copy(pk_hbm.at[wid, ci], pk_c.at[b],
                                  sem_i[b]).wait()

            @pl.when(it > 0)
            def _():
                pltpu.make_async_copy(
                    rows_v.at[b], acc_sh.at[di_c.at[b]], sem_s[b]).wait()
                pltpu.make_async_copy(
                    pv_v.at[b], den_sh.at[di_c.at[b]], sem_d[b]).wait()

            @pl.loop(0, _CH // 16)
            def _(j):
                sl = pl.ds(j * 16, 16)
                pk16 = pk_c[b, sl]
                si_c[b, sl] = pk16 & msk
                di_c[b, sl] = jnp.right_shift(pk16, 16)

            pltpu.async_copy(h_hbm.at[si_c.at[b]], rows_v.at[b], sem_g[b])

            @pl.when(it < _NCH // 2 - 1)
            def _():
                pltpu.async_copy(pk_hbm.at[wid, ci + 2], pk_c.at[b],
                                 sem_i[b])

            @pl.loop(0, _CH // 16)
            def _(j):
                sl = pl.ds(j * 16, 16)
                iv = si_c[b, sl]
                dv = jnp.minimum(di_c[b, sl], _NP - 1)
                e = (plsc.load_gather(ev_v, [iv + iv])
                     + plsc.load_gather(ev_v, [dv + dv + one16]))
                e = jnp.where(e >= 0.0, e, 0.2 * e)
                pv_v[b, sl] = jnp.exp(e - m16)

            pltpu.make_async_copy(h_hbm.at[si_c.at[b]], rows_v.at[b],
                                  sem_g[b]).wait()

            b16 = jnp.full((16,), b, jnp.int32)

            @pl.loop(0, _CH)
            def _(r):
                pb = plsc.load_gather(pv_v, [b16, jnp.full((16,), r,
                                                           jnp.int32)])
                for c in range(8):
                    sl = pl.ds(c * 16, 16)
                    rows_v[b, r, sl] = rows_v[b, r, sl] * pb

            pltpu.async_copy(rows_v.at[b], acc_sh.at[di_c.at[b]],
                             sem_s[b], add=True)
            pltpu.async_copy(pv_v.at[b], den_sh.at[di_c.at[b]],
                             sem_d[b], add=True)

    for b in range(2):
        pltpu.make_async_copy(rows_v.at[b], acc_sh.at[di_c.at[b]],
                              sem_s[b]).wait()
        pltpu.make_async_copy(pv_v.at[b], den_sh.at[di_c.at[b]],
                              sem_d[b]).wait()

    plsc.subcore_barrier()
    out_sl = pl.ds(base, _NP // 16)
    pltpu.sync_copy(acc_sh.at[out_sl], p_out.at[cid, out_sl])
    pltpu.sync_copy(den_sh.at[out_sl], den_out.at[cid, out_sl])


_SC_PARAMS = pltpu.CompilerParams()
if "needs_layout_passes" in pltpu.CompilerParams.__dataclass_fields__:
    _SC_PARAMS = dataclasses.replace(_SC_PARAMS, needs_layout_passes=False)


def _sc_edge(h, ev, m, pk3):
    kern = pl.kernel(
        _edge_body,
        compiler_params=_SC_PARAMS,
        out_type=[jax.ShapeDtypeStruct((2, _NP, _F), jnp.float32),
                  jax.ShapeDtypeStruct((2, _NP), jnp.float32)],
        mesh=plsc.VectorSubcoreMesh(core_axis_name="c", subcore_axis_name="s"),
        scratch_types=[
            pltpu.VMEM((2 * _NP,), jnp.float32),
            pltpu.VMEM((16,), jnp.float32),
            pltpu.VMEM((2, _CH), jnp.int32),
            pltpu.VMEM((2, _CH), jnp.int32),
            pltpu.VMEM((2, _CH), jnp.int32),
            pltpu.VMEM((2, _CH, _F), jnp.float32),
            pltpu.VMEM((2, _CH), jnp.float32),
            pltpu.VMEM_SHARED((_AR, _F), jnp.float32),
            pltpu.VMEM_SHARED((_AR,), jnp.float32),
        ] + [pltpu.SemaphoreType.DMA] * 8,
    )
    return kern(h, ev, m, pk3)


def _pre_body(x_ref, w_ref, av_ref, h_ref, ev_ref, m_ref):
    h = jnp.dot(x_ref[...], w_ref[...], precision=_HI)
    h_ref[...] = h
    ev = jnp.dot(h, av_ref[...], precision=_HI)
    ev_ref[...] = ev
    mx = jnp.max(ev, axis=0, keepdims=True)
    m = _lrelu(mx[0:1, 0:1] + mx[0:1, 1:2])
    m_ref[...] = jnp.broadcast_to(m, (8, 128))


def _tc_pre(x, w, av):
    return pl.pallas_call(
        _pre_body,
        out_shape=[jax.ShapeDtypeStruct((_NP, _F), jnp.float32),
                   jax.ShapeDtypeStruct((_NP, 2), jnp.float32),
                   jax.ShapeDtypeStruct((8, 128), jnp.float32)],
    )(x, w, av)


def _combine(p_ref, d_ref, ev_ref, m_ref, h_ref, b_ref):
    es = ev_ref[:, 0:1]
    ed = ev_ref[:, 1:2]
    m = m_ref[0:1, 0:1]
    ps = jnp.exp(_lrelu(es + ed) - m)
    h = h_ref[...]
    num = p_ref[0] + p_ref[1] + ps * h
    den = d_ref[0] + d_ref[1] + ps + 1e-16
    return num / den + b_ref[...]


def _mid_body(p_ref, d_ref, ev_ref, m_ref, h_ref, b_ref, w_ref, av_ref,
              h2_ref, ev2_ref, m2_ref):
    a = jnp.maximum(_combine(p_ref, d_ref, ev_ref, m_ref, h_ref, b_ref), 0.0)
    h2 = jnp.dot(a, w_ref[...], precision=_HI)
    h2_ref[...] = h2
    ev2 = jnp.dot(h2, av_ref[...], precision=_HI)
    ev2_ref[...] = ev2
    mx = jnp.max(ev2, axis=0, keepdims=True)
    m2 = _lrelu(mx[0:1, 0:1] + mx[0:1, 1:2])
    m2_ref[...] = jnp.broadcast_to(m2, (8, 128))


def _tc_mid(p, d, ev, m, h, b, w, av):
    return pl.pallas_call(
        _mid_body,
        out_shape=[jax.ShapeDtypeStruct((_NP, _F), jnp.float32),
                   jax.ShapeDtypeStruct((_NP, 2), jnp.float32),
                   jax.ShapeDtypeStruct((8, 128), jnp.float32)],
    )(p, d, ev, m, h, b, w, av)


def _fin_body(p_ref, d_ref, ev_ref, m_ref, h_ref, b_ref, batch_ref,
              wl_ref, bl_ref, out_ref):
    o = _combine(p_ref, d_ref, ev_ref, m_ref, h_ref, b_ref)
    bb = jnp.broadcast_to(batch_ref[...], (_G, _NP))
    gid = lax.broadcasted_iota(jnp.int32, (_G, _NP), 0)
    oh = (bb == gid).astype(jnp.float32)
    sums = jnp.dot(oh, o, precision=_HI)
    cnt = jnp.sum(oh, axis=1, keepdims=True)
    pooled = sums / jnp.maximum(cnt, 1.0)
    out_ref[...] = jnp.dot(pooled, wl_ref[...], precision=_HI) + bl_ref[...]


def _tc_fin(p, d, ev, m, h, b, batch2, wl, bl):
    return pl.pallas_call(
        _fin_body,
        out_shape=jax.ShapeDtypeStruct((_G, _F), jnp.float32),
    )(p, d, ev, m, h, b, batch2, wl, bl)


def kernel(x, edge_index, batch, W1, a_src1, a_dst1, b1, W2, a_src2, a_dst2,
           b2, W3, a_src3, a_dst3, b3, Wlin, blin):
    x_p = jnp.pad(x, ((0, _NP - _N), (0, 0)))
    src = edge_index[0]
    dst = edge_index[1]
    npad = _EP - _E
    src_p = jnp.concatenate([src, jnp.zeros((npad,), jnp.int32)])
    dst_p = jnp.concatenate(
        [dst, _NP + (jnp.arange(npad, dtype=jnp.int32) % _DUMP)])
    pk3 = (src_p | (dst_p << 16)).reshape(_NW, _NCH, _CH)
    batch2 = jnp.concatenate(
        [batch, jnp.full((_NP - _N,), _G, jnp.int32)]).reshape(1, _NP)

    av1 = jnp.stack([a_src1, a_dst1], axis=1)
    av2 = jnp.stack([a_src2, a_dst2], axis=1)
    av3 = jnp.stack([a_src3, a_dst3], axis=1)

    h1, ev1, m1 = _tc_pre(x_p, W1, av1)
    P1, D1 = _sc_edge(h1, ev1.reshape(2 * _NP), m1, pk3)
    h2, ev2, m2 = _tc_mid(P1, D1.reshape(2, _NP, 1), ev1, m1, h1,
                          b1.reshape(1, _F), W2, av2)
    P2, D2 = _sc_edge(h2, ev2.reshape(2 * _NP), m2, pk3)
    h3, ev3, m3 = _tc_mid(P2, D2.reshape(2, _NP, 1), ev2, m2, h2,
                          b2.reshape(1, _F), W3, av3)
    P3, D3 = _sc_edge(h3, ev3.reshape(2 * _NP), m3, pk3)
    return _tc_fin(P3, D3.reshape(2, _NP, 1), ev3, m3, h3,
                   b3.reshape(1, _F), batch2, Wlin, blin.reshape(1, _F))
